# baseline (speedup 1.0000x reference)
import functools

import jax
import jax.numpy as jnp
from jax import lax
from jax.experimental import pallas as pl
from jax.experimental.pallas import tpu as pltpu
from jax.experimental.pallas import tpu_sc as plsc

_BATCH = 1024
_FEAT = 64
_NUM_CLASSES = 100000
_LANES = 16

_NC = 2
_NS = 16
_NW = _NC * _NS
_TW = 128
_GRP = 4

_TC_N = 512
_SC_N = _BATCH - _TC_N
_BPW = _SC_N // _NW


@functools.partial(
    pl.kernel,
    mesh=plsc.VectorSubcoreMesh(core_axis_name="c", subcore_axis_name="s"),
    compiler_params=pltpu.CompilerParams(
        needs_layout_passes=False, skip_device_barrier=True),
    out_type=jax.ShapeDtypeStruct((_NW, _LANES), jnp.float32),
    scratch_types=[
        pltpu.VMEM((_BPW,), jnp.int32),
        pltpu.VMEM((_BPW, _FEAT), jnp.float32),
        pltpu.VMEM((2 * _GRP, _FEAT, _TW), jnp.float32),
        pltpu.VMEM((_LANES,), jnp.float32),
        pltpu.SemaphoreType.DMA,
        pltpu.SemaphoreType.DMA,
        pltpu.SemaphoreType.DMA,
    ],
)
def _center_loss_sc(feat_hbm, tgt_hbm, ct_hbm, out_hbm,
                    idx_v, f_v, slab_v, o_v, fsem, gsem_a, gsem_b):
    wid = lax.axis_index("s") * _NC + lax.axis_index("c")
    base = _TC_N + wid * _BPW
    pltpu.sync_copy(tgt_hbm.at[pl.ds(base, _BPW)], idx_v)
    fcp = pltpu.async_copy(feat_hbm.at[pl.ds(base, _BPW)], f_v, fsem)

    tgt_rows = [None] * _BPW
    for g in range(_BPW // _LANES):
        tv = idx_v[pl.ds(g * _LANES, _LANES)]
        for i in range(_LANES):
            tgt_rows[g * _LANES + i] = tv[i]

    sems = [gsem_a, gsem_b]
    nbatch = _BPW // _GRP

    def fire_batch(b):
        cps = []
        for k in range(_GRP):
            r = tgt_rows[b * _GRP + k]
            col0 = pl.multiple_of((r // _TW) * _TW, _TW)
            cps.append(
                pltpu.async_copy(ct_hbm.at[:, pl.ds(col0, _TW)],
                                 slab_v.at[(b % 2) * _GRP + k],
                                 sems[b % 2]))
        return cps

    lanes = lax.iota(jnp.int32, _LANES)
    fcp.wait()
    inflight = fire_batch(0)
    acc = jnp.zeros((_LANES,), jnp.float32)
    for b in range(nbatch):
        nxt = fire_batch(b + 1) if b + 1 < nbatch else []
        for cp in inflight:
            cp.wait()
        for k in range(_GRP):
            i = b * _GRP + k
            r = tgt_rows[i]
            cloc = jnp.full((_LANES,), r % _TW, jnp.int32)
            sbuf = slab_v.at[(b % 2) * _GRP + k]
            for ch in range(_FEAT // _LANES):
                dims = lanes + jnp.int32(ch * _LANES)
                cvals = plsc.load_gather(sbuf, [dims, cloc])
                df = f_v[i, pl.ds(ch * _LANES, _LANES)] - cvals
                acc = acc + df * df
        inflight = nxt
    o_v[...] = acc
    pltpu.sync_copy(o_v, out_hbm.at[wid])


def _center_loss_tc_body(tgt_ref, slab_ref, frow_ref, cacc_ref, facc_ref):
    i = pl.program_id(0)

    @pl.when(i == 0)
    def _init():
        cacc_ref[...] = jnp.zeros_like(cacc_ref)
        facc_ref[...] = jnp.zeros_like(facc_ref)

    col = tgt_ref[i] % _TW
    slab = slab_ref[...]
    frow = frow_ref[pl.ds(i % 8, 1), :]
    ones = jnp.ones((1, _FEAT), jnp.float32)
    nsq = jnp.dot(ones, slab * slab,
                  preferred_element_type=jnp.float32)
    m2 = jnp.dot(frow, slab,
                 preferred_element_type=jnp.float32)
    onehot = (lax.broadcasted_iota(jnp.int32, (1, _TW), 1) == col)
    sel = jnp.where(onehot, nsq - 2.0 * m2, 0.0)
    cacc_ref[...] += sel
    facc_ref[...] += frow * frow


def _center_loss_tc(features, target, ct):
    grid_spec = pltpu.PrefetchScalarGridSpec(
        num_scalar_prefetch=1,
        grid=(_TC_N,),
        in_specs=[
            pl.BlockSpec((_FEAT, _TW), lambda i, t: (0, t[i] // _TW)),
            pl.BlockSpec((8, _FEAT), lambda i, t: (i // 8, 0)),
        ],
        out_specs=[
            pl.BlockSpec((1, _TW), lambda i, t: (0, 0)),
            pl.BlockSpec((1, _FEAT), lambda i, t: (0, 0)),
        ],
    )
    return pl.pallas_call(
        _center_loss_tc_body,
        grid_spec=grid_spec,
        out_shape=[
            jax.ShapeDtypeStruct((1, _TW), jnp.float32),
            jax.ShapeDtypeStruct((1, _FEAT), jnp.float32),
        ],
        compiler_params=pltpu.CompilerParams(
            dimension_semantics=("arbitrary",)),
    )(target, ct, features)


def kernel(features, target, centers):
    ct = centers.T
    sc_partials = _center_loss_sc(features, target, ct)
    tc_cacc, tc_facc = _center_loss_tc(features, target, ct)
    zero_term = jnp.float32((_NUM_CLASSES - 1) * 1e-12)
    total = jnp.sum(sc_partials) + jnp.sum(tc_cacc) + jnp.sum(tc_facc)
    return total / jnp.float32(_BATCH) + zero_term


# speedup vs baseline: 6.5782x; 6.5782x over previous
import functools

import jax
import jax.numpy as jnp
from jax import lax
from jax.experimental import pallas as pl
from jax.experimental.pallas import tpu as pltpu
from jax.experimental.pallas import tpu_sc as plsc

_BATCH = 1024
_FEAT = 64
_NUM_CLASSES = 100000
_LANES = 16

_NC = 2
_NS = 16
_NW = _NC * _NS
_BPW = _BATCH // _NW
_TW = 128
_GRP = 4


@functools.partial(
    pl.kernel,
    mesh=plsc.VectorSubcoreMesh(core_axis_name="c", subcore_axis_name="s"),
    compiler_params=pltpu.CompilerParams(
        needs_layout_passes=False, skip_device_barrier=True),
    out_type=jax.ShapeDtypeStruct((_NW, _LANES), jnp.float32),
    scratch_types=[
        pltpu.VMEM((_BPW,), jnp.int32),
        pltpu.VMEM((_FEAT, _TW), jnp.float32),
        pltpu.VMEM((2 * _GRP, _FEAT, _TW), jnp.float32),
        pltpu.VMEM((_LANES,), jnp.float32),
        pltpu.SemaphoreType.DMA,
        pltpu.SemaphoreType.DMA,
        pltpu.SemaphoreType.DMA,
    ],
)
def _center_loss_sc(ft_hbm, tgt_hbm, ct_hbm, out_hbm,
                    idx_v, f_v, slab_v, o_v, fsem, gsem_a, gsem_b):
    wid = lax.axis_index("s") * _NC + lax.axis_index("c")
    base = wid * _BPW
    col_off = (wid % (_TW // _BPW)) * _BPW
    pltpu.sync_copy(tgt_hbm.at[pl.ds(base, _BPW)], idx_v)
    fblk = pl.multiple_of((wid // (_TW // _BPW)) * _TW, _TW)
    fcp = pltpu.async_copy(ft_hbm.at[:, pl.ds(fblk, _TW)], f_v, fsem)

    tgt_rows = [None] * _BPW
    for g in range(_BPW // _LANES):
        tv = idx_v[pl.ds(g * _LANES, _LANES)]
        for i in range(_LANES):
            tgt_rows[g * _LANES + i] = tv[i]

    sems = [gsem_a, gsem_b]
    nbatch = _BPW // _GRP

    def fire_batch(b):
        cps = []
        for k in range(_GRP):
            r = tgt_rows[b * _GRP + k]
            col0 = pl.multiple_of((r // _TW) * _TW, _TW)
            cps.append(
                pltpu.async_copy(ct_hbm.at[:, pl.ds(col0, _TW)],
                                 slab_v.at[(b % 2) * _GRP + k],
                                 sems[b % 2]))
        return cps

    lanes = lax.iota(jnp.int32, _LANES)
    fcp.wait()
    inflight = fire_batch(0)
    acc = jnp.zeros((_LANES,), jnp.float32)
    for b in range(nbatch):
        nxt = fire_batch(b + 1) if b + 1 < nbatch else []
        for cp in inflight:
            cp.wait()
        for k in range(_GRP):
            i = b * _GRP + k
            r = tgt_rows[i]
            cloc = jnp.full((_LANES,), r % _TW, jnp.int32)
            floc = jnp.full((_LANES,), col_off + i, jnp.int32)
            sbuf = slab_v.at[(b % 2) * _GRP + k]
            for ch in range(_FEAT // _LANES):
                dims = lanes + jnp.int32(ch * _LANES)
                cvals = plsc.load_gather(sbuf, [dims, cloc])
                fvals = plsc.load_gather(f_v, [dims, floc])
                df = fvals - cvals
                acc = acc + df * df
        inflight = nxt
    o_v[...] = acc
    pltpu.sync_copy(o_v, out_hbm.at[wid])


def kernel(features, target, centers):
    partials = _center_loss_sc(features.T, target, centers.T)
    zero_term = jnp.float32((_NUM_CLASSES - 1) * 1e-12)
    return jnp.sum(partials) / jnp.float32(_BATCH) + zero_term


# speedup vs baseline: 7.0018x; 1.0644x over previous
import functools

import jax
import jax.numpy as jnp
from jax import lax
from jax.experimental import pallas as pl
from jax.experimental.pallas import tpu as pltpu
from jax.experimental.pallas import tpu_sc as plsc

_BATCH = 1024
_FEAT = 64
_NUM_CLASSES = 100000
_LANES = 16

_NC = 2
_NS = 16
_NW = _NC * _NS
_BPW = _BATCH // _NW
_TW = 128
_GRP = 4


@functools.partial(
    pl.kernel,
    mesh=plsc.VectorSubcoreMesh(core_axis_name="c", subcore_axis_name="s"),
    compiler_params=pltpu.CompilerParams(
        needs_layout_passes=False, skip_device_barrier=True),
    out_type=jax.ShapeDtypeStruct((_NW, _LANES), jnp.float32),
    scratch_types=[
        pltpu.VMEM((_BPW,), jnp.int32),
        pltpu.VMEM((_BPW, _FEAT), jnp.float32),
        pltpu.VMEM((3 * _GRP, _FEAT, _TW), jnp.float32),
        pltpu.VMEM((_LANES,), jnp.float32),
        pltpu.SemaphoreType.DMA,
        pltpu.SemaphoreType.DMA,
        pltpu.SemaphoreType.DMA,
        pltpu.SemaphoreType.DMA,
    ],
)
def _center_loss_partials(feat_hbm, tgt_hbm, ct_hbm, out_hbm,
                          idx_v, f_v, slab_v, o_v, fsem,
                          gsem_a, gsem_b, gsem_c):
    wid = lax.axis_index("s") * _NC + lax.axis_index("c")
    base = wid * _BPW
    pltpu.sync_copy(tgt_hbm.at[pl.ds(base, _BPW)], idx_v)
    fcp = pltpu.async_copy(feat_hbm.at[pl.ds(base, _BPW)], f_v, fsem)

    tgt_rows = [None] * _BPW
    for g in range(_BPW // _LANES):
        tv = idx_v[pl.ds(g * _LANES, _LANES)]
        for i in range(_LANES):
            tgt_rows[g * _LANES + i] = tv[i]

    sems = [gsem_a, gsem_b, gsem_c]
    nbatch = _BPW // _GRP
    _D = 3

    def fire_batch(b):
        cps = []
        for k in range(_GRP):
            r = tgt_rows[b * _GRP + k]
            col0 = pl.multiple_of((r // _TW) * _TW, _TW)
            cps.append(
                pltpu.async_copy(ct_hbm.at[:, pl.ds(col0, _TW)],
                                 slab_v.at[(b % _D) * _GRP + k],
                                 sems[b % _D]))
        return cps

    lanes = lax.iota(jnp.int32, _LANES)
    fcp.wait()
    inflight = [fire_batch(0), fire_batch(1)]
    acc = jnp.zeros((_LANES,), jnp.float32)
    for b in range(nbatch):
        if b + 2 < nbatch:
            inflight.append(fire_batch(b + 2))
        for cp in inflight.pop(0):
            cp.wait()
        for k in range(_GRP):
            i = b * _GRP + k
            r = tgt_rows[i]
            cloc = jnp.full((_LANES,), r % _TW, jnp.int32)
            sbuf = slab_v.at[(b % _D) * _GRP + k]
            for ch in range(_FEAT // _LANES):
                dims = lanes + jnp.int32(ch * _LANES)
                cvals = plsc.load_gather(sbuf, [dims, cloc])
                df = f_v[i, pl.ds(ch * _LANES, _LANES)] - cvals
                acc = acc + df * df
    o_v[...] = acc
    pltpu.sync_copy(o_v, out_hbm.at[wid])


def kernel(features, target, centers):
    partials = _center_loss_partials(features, target, centers.T)
    zero_term = jnp.float32((_NUM_CLASSES - 1) * 1e-12)
    return jnp.sum(partials) / jnp.float32(_BATCH) + zero_term
